# baseline (speedup 1.0000x reference)
import jax
import jax.numpy as jnp
from jax.experimental import pallas as pl
from jax.experimental.pallas import tpu as pltpu

_BK = 256

def _router_body(x_ref, w_ref, out_ref, acc_ref):
    i = pl.program_id(0)
    nk = pl.num_programs(0)
    partial = jax.lax.dot_general(
        w_ref[...],
        x_ref[...],
        dimension_numbers=(((1,), (1,)), ((), ())),
        preferred_element_type=jnp.float32,
    )

    @pl.when(i == 0)
    def _():
        acc_ref[...] = partial

    @pl.when(i > 0)
    def _():
        acc_ref[...] += partial

    @pl.when(i == nk - 1)
    def _():
        out_ref[...] = jnp.swapaxes(acc_ref[...], 0, 1)

def kernel(x, W):
    m, k = x.shape
    e = W.shape[0]
    return pl.pallas_call(
        _router_body,
        grid=(k // _BK,),
        in_specs=[
            pl.BlockSpec((m, _BK), lambda i: (0, i)),
            pl.BlockSpec((e, _BK), lambda i: (0, i)),
        ],
        out_specs=pl.BlockSpec((m, e), lambda i: (0, 0)),
        out_shape=jax.ShapeDtypeStruct((m, e), jnp.float32),
        scratch_shapes=[pltpu.VMEM((e, m), jnp.float32)],
    )(x, W)


# speedup vs baseline: 1.0885x; 1.0885x over previous
import jax
import jax.numpy as jnp
from jax.experimental import pallas as pl
from jax.experimental.pallas import tpu as pltpu


_BM = 1024


def _router_body(x_ref, w_ref, out_hbm, otile, sems):
    i = pl.program_id(0)
    n = pl.num_programs(0)
    b = jax.lax.rem(i, 2)

    def _copy(step, buf):
        return pltpu.make_async_copy(
            otile.at[buf],
            out_hbm.at[pl.ds(step * _BM, _BM), :],
            sems.at[buf],
        )

    @pl.when(i >= 2)
    def _():
        _copy(i - 2, b).wait()

    otile[b] = jax.lax.dot_general(
        x_ref[...],
        w_ref[...],
        dimension_numbers=(((1,), (1,)), ((), ())),
        preferred_element_type=jnp.float32,
    )
    _copy(i, b).start()

    @pl.when(i == n - 1)
    def _():
        _copy(i - 1, 1 - b).wait()
        _copy(i, b).wait()


def kernel(x, W):
    m, k = x.shape
    e = W.shape[0]
    return pl.pallas_call(
        _router_body,
        grid=(m // _BM,),
        in_specs=[
            pl.BlockSpec((_BM, k), lambda i: (i, 0)),
            pl.BlockSpec((e, k), lambda i: (0, 0)),
        ],
        out_specs=pl.BlockSpec(memory_space=pl.ANY),
        out_shape=jax.ShapeDtypeStruct((m, e), jnp.float32),
        scratch_shapes=[
            pltpu.VMEM((2, _BM, e), jnp.float32),
            pltpu.SemaphoreType.DMA((2,)),
        ],
    )(x, W)


# speedup vs baseline: 1.2739x; 1.1703x over previous
import jax
import jax.numpy as jnp
from jax.experimental import pallas as pl


_BM = 1024


def _router_body(x_ref, w_ref, out_ref):
    out_ref[...] = jax.lax.dot_general(
        w_ref[...],
        x_ref[...],
        dimension_numbers=(((1,), (1,)), ((), ())),
        preferred_element_type=jnp.float32,
    )


def kernel(x, W):
    m, k = x.shape
    e = W.shape[0]
    out_t = pl.pallas_call(
        _router_body,
        grid=(m // _BM,),
        in_specs=[
            pl.BlockSpec((_BM, k), lambda i: (i, 0)),
            pl.BlockSpec((e, k), lambda i: (0, 0)),
        ],
        out_specs=pl.BlockSpec((e, _BM), lambda i: (0, i)),
        out_shape=jax.ShapeDtypeStruct((e, m), jnp.float32),
    )(x, W)
    return out_t.T
